# baseline (speedup 1.0000x reference)
import jax
import jax.numpy as jnp
from jax.experimental import pallas as pl
from jax.experimental.pallas import tpu as pltpu

NT = 4096
NCOL = 384 * 3
BLK = 128
NBLK = NCOL // BLK


def _shift(x, j):
    m = x.shape[0]
    if j > 0:
        return jnp.concatenate([x[j:], x[:j]], axis=0)
    j = -j
    return jnp.concatenate([x[m - j:], x[:m - j]], axis=0)


def _uniform_large(x, j, min_down):
    m = x.shape[0]
    x4 = x.reshape(m // (2 * j), 2, j, BLK)
    a = x4[:, 0]
    b = x4[:, 1]
    lo = jnp.minimum(a, b)[:, None]
    hi = jnp.maximum(a, b)[:, None]
    pair = [lo, hi] if min_down else [hi, lo]
    return jnp.concatenate(pair, axis=1).reshape(m, BLK)


def _uniform_small(x, j, masks, min_down):
    bitj = masks[(x.shape[0], j)]
    u = _shift(x, j)
    d = _shift(x, -j)
    if min_down:
        return jnp.where(bitj, jnp.maximum(x, d), jnp.minimum(x, u))
    return jnp.where(bitj, jnp.minimum(x, d), jnp.maximum(x, u))


def _masked_substage(x, j, bitj, keep_min):
    p = jnp.where(bitj, _shift(x, -j), _shift(x, j))
    return jnp.where(keep_min, jnp.minimum(x, p), jnp.maximum(x, p))


def _stage_uniform(x, k, masks, min_down):
    j = k // 2
    while j >= 1:
        if j >= 8:
            x = _uniform_large(x, j, min_down)
        else:
            x = _uniform_small(x, j, masks, min_down)
        j //= 2
    return x


def _tile_sort64(x, masks, ascending):
    for k in (2, 4):
        kbit = masks[("k", k)]
        j = k // 2
        while j >= 1:
            bitj = masks[(64, j)]
            keep_min = (bitj == kbit) if ascending else (bitj != kbit)
            x = _masked_substage(x, j, bitj, keep_min)
            j //= 2
    for k in (8, 16, 32):
        g = 64 // (2 * k)
        x6 = x.reshape(g, 2, k, BLK)
        xa = x6[:, 0].reshape(32, BLK)
        xd = x6[:, 1].reshape(32, BLK)
        xa = _stage_uniform(xa, k, masks, ascending)
        xd = _stage_uniform(xd, k, masks, not ascending)
        x = jnp.concatenate(
            [xa.reshape(g, 1, k, BLK), xd.reshape(g, 1, k, BLK)],
            axis=1).reshape(64, BLK)
    return _stage_uniform(x, 64, masks, ascending)


def _wasserstein_kernel(pred_ref, obs_ref, out_ref, a_s, b_s):
    masks = {}
    for rows in (32, 64):
        it = jax.lax.broadcasted_iota(jnp.int32, (rows, 1), 0)
        for j in (1, 2, 4):
            masks[(rows, j)] = (it & j) != 0
        if rows == 64:
            for k in (2, 4, 8):
                masks[("k", k)] = (it & k) != 0

    def body1(m, carry):
        for src, dst in ((pred_ref, a_s), (obs_ref, b_s)):
            for off, asc in ((m * 128, True), (m * 128 + 64, False)):
                xt = src[pl.ds(off, 64), :]
                dst[pl.ds(off, 64), :] = _tile_sort64(xt, masks, asc)
        return carry

    jax.lax.fori_loop(0, 32, body1, 0, unroll=8)

    k = 128
    while k <= NT:
        for ref in (a_s, b_s):
            x = ref[...]
            if k < NT:
                x6 = x.reshape(NT // (2 * k), 2, k, BLK)
                xa = x6[:, 0].reshape(NT // 2, BLK)
                xd = x6[:, 1].reshape(NT // 2, BLK)
                j = k // 2
                while j >= 64:
                    xa = _uniform_large(xa, j, True)
                    xd = _uniform_large(xd, j, False)
                    j //= 2
                x = jnp.concatenate(
                    [xa.reshape(NT // (2 * k), 1, k, BLK),
                     xd.reshape(NT // (2 * k), 1, k, BLK)],
                    axis=1).reshape(NT, BLK)
            else:
                j = k // 2
                while j >= 64:
                    x = _uniform_large(x, j, True)
                    j //= 2
            ref[...] = x

        q2 = k // 64

        if k < NT:
            def tail_body(m, carry, q2=q2):
                t_asc = (m // q2) * (2 * q2) + (m % q2)
                for ref in (a_s, b_s):
                    for t, asc in ((t_asc, True), (t_asc + q2, False)):
                        off = t * 64
                        xt = ref[pl.ds(off, 64), :]
                        ref[pl.ds(off, 64), :] = _stage_uniform(
                            xt, 64, masks, asc)
                return carry
        else:
            def tail_body(m, carry, q2=q2):
                for t in (2 * m, 2 * m + 1):
                    off = t * 64
                    xa = _stage_uniform(a_s[pl.ds(off, 64), :], 64, masks, True)
                    xb = _stage_uniform(b_s[pl.ds(off, 64), :], 64, masks, True)
                    carry = carry + jnp.sum(jnp.abs(xa - xb), axis=0,
                                            keepdims=True)
                return carry

        if k < NT:
            jax.lax.fori_loop(0, 32, tail_body, 0, unroll=8)
        else:
            total = jax.lax.fori_loop(
                0, 32, tail_body, jnp.zeros((1, BLK), jnp.float32), unroll=8)
            out_ref[0, 0, :] = total[0]
        k *= 2


@jax.jit
def kernel(pred_waveforms, obs_waveforms):
    pred = pred_waveforms.reshape(NT, NCOL)
    obs = obs_waveforms.reshape(NT, NCOL)
    partial = pl.pallas_call(
        _wasserstein_kernel,
        grid=(NBLK,),
        in_specs=[
            pl.BlockSpec((NT, BLK), lambda i: (0, i)),
            pl.BlockSpec((NT, BLK), lambda i: (0, i)),
        ],
        out_specs=pl.BlockSpec((1, 1, BLK), lambda i: (i, 0, 0)),
        out_shape=jax.ShapeDtypeStruct((NBLK, 1, BLK), jnp.float32),
        scratch_shapes=[
            pltpu.VMEM((NT, BLK), jnp.float32),
            pltpu.VMEM((NT, BLK), jnp.float32),
        ],
    )(pred, obs)
    return jnp.sum(partial) / (NT * NCOL)


# speedup vs baseline: 1.0070x; 1.0070x over previous
import jax
import jax.numpy as jnp
from jax.experimental import pallas as pl
from jax.experimental.pallas import tpu as pltpu

NT = 4096
NCOL = 384 * 3
BLK = 128
NBLK = NCOL // BLK


def _shift(x, j):
    m = x.shape[0]
    if j > 0:
        return jnp.concatenate([x[j:], x[:j]], axis=0)
    j = -j
    return jnp.concatenate([x[m - j:], x[:m - j]], axis=0)


def _uniform_large(x, j, min_down):
    m = x.shape[0]
    x4 = x.reshape(m // (2 * j), 2, j, BLK)
    a = x4[:, 0]
    b = x4[:, 1]
    lo = jnp.minimum(a, b)[:, None]
    hi = jnp.maximum(a, b)[:, None]
    pair = [lo, hi] if min_down else [hi, lo]
    return jnp.concatenate(pair, axis=1).reshape(m, BLK)


def _uniform_small(x, j, masks, min_down):
    bitj = masks[(x.shape[0], j)]
    u = _shift(x, j)
    d = _shift(x, -j)
    if min_down:
        return jnp.where(bitj, jnp.maximum(x, d), jnp.minimum(x, u))
    return jnp.where(bitj, jnp.minimum(x, d), jnp.maximum(x, u))


def _masked_substage(x, j, bitj, keep_min):
    p = jnp.where(bitj, _shift(x, -j), _shift(x, j))
    return jnp.where(keep_min, jnp.minimum(x, p), jnp.maximum(x, p))


def _stage_uniform(x, k, masks, min_down):
    j = k // 2
    while j >= 1:
        if j >= 8:
            x = _uniform_large(x, j, min_down)
        else:
            x = _uniform_small(x, j, masks, min_down)
        j //= 2
    return x


def _tile_sort64(x, masks, ascending):
    for k in (2, 4):
        kbit = masks[("k", k)]
        j = k // 2
        while j >= 1:
            bitj = masks[(64, j)]
            keep_min = (bitj == kbit) if ascending else (bitj != kbit)
            x = _masked_substage(x, j, bitj, keep_min)
            j //= 2
    for k in (8, 16, 32):
        g = 64 // (2 * k)
        x6 = x.reshape(g, 2, k, BLK)
        xa = x6[:, 0].reshape(32, BLK)
        xd = x6[:, 1].reshape(32, BLK)
        xa = _stage_uniform(xa, k, masks, ascending)
        xd = _stage_uniform(xd, k, masks, not ascending)
        x = jnp.concatenate(
            [xa.reshape(g, 1, k, BLK), xd.reshape(g, 1, k, BLK)],
            axis=1).reshape(64, BLK)
    return _stage_uniform(x, 64, masks, ascending)


def _wasserstein_kernel(pred_ref, obs_ref, out_ref, a_s, b_s):
    masks = {}
    for rows in (32, 64):
        it = jax.lax.broadcasted_iota(jnp.int32, (rows, 1), 0)
        for j in (1, 2, 4):
            masks[(rows, j)] = (it & j) != 0
        if rows == 64:
            for k in (2, 4, 8):
                masks[("k", k)] = (it & k) != 0

    def body1(m, carry):
        for src, dst in ((pred_ref, a_s), (obs_ref, b_s)):
            for off, asc in ((m * 128, True), (m * 128 + 64, False)):
                xt = src[pl.ds(off, 64), :]
                dst[pl.ds(off, 64), :] = _tile_sort64(xt, masks, asc)
        return carry

    jax.lax.fori_loop(0, 32, body1, 0, unroll=4)

    def _pair_tail(ref, p, asc):
        off = p * 128
        t0 = ref[pl.ds(off, 64), :]
        t1 = ref[pl.ds(off + 64, 64), :]
        lo = jnp.minimum(t0, t1)
        hi = jnp.maximum(t0, t1)
        x0, x1 = (lo, hi) if asc else (hi, lo)
        ref[pl.ds(off, 64), :] = _stage_uniform(x0, 64, masks, asc)
        ref[pl.ds(off + 64, 64), :] = _stage_uniform(x1, 64, masks, asc)

    k = 128
    while k <= NT:
        if k > 128:
            for ref in (a_s, b_s):
                x = ref[...]
                if k < NT:
                    x6 = x.reshape(NT // (2 * k), 2, k, BLK)
                    xa = x6[:, 0].reshape(NT // 2, BLK)
                    xd = x6[:, 1].reshape(NT // 2, BLK)
                    j = k // 2
                    while j >= 128:
                        xa = _uniform_large(xa, j, True)
                        xd = _uniform_large(xd, j, False)
                        j //= 2
                    x = jnp.concatenate(
                        [xa.reshape(NT // (2 * k), 1, k, BLK),
                         xd.reshape(NT // (2 * k), 1, k, BLK)],
                        axis=1).reshape(NT, BLK)
                else:
                    j = k // 2
                    while j >= 128:
                        x = _uniform_large(x, j, True)
                        j //= 2
                ref[...] = x

        q = k // 128

        if k < NT:
            def tail_body(m, carry, q=q):
                p_asc = (m // q) * (2 * q) + (m % q)
                for ref in (a_s, b_s):
                    _pair_tail(ref, p_asc, True)
                    _pair_tail(ref, p_asc + q, False)
                return carry

            jax.lax.fori_loop(0, 16, tail_body, 0, unroll=2)
        else:
            def tail_body(p, carry):
                off = p * 128
                outs = []
                for ref in (a_s, b_s):
                    t0 = ref[pl.ds(off, 64), :]
                    t1 = ref[pl.ds(off + 64, 64), :]
                    lo = jnp.minimum(t0, t1)
                    hi = jnp.maximum(t0, t1)
                    outs.append((_stage_uniform(lo, 64, masks, True),
                                 _stage_uniform(hi, 64, masks, True)))
                (a0, a1), (b0, b1) = outs
                return (carry
                        + jnp.sum(jnp.abs(a0 - b0), axis=0, keepdims=True)
                        + jnp.sum(jnp.abs(a1 - b1), axis=0, keepdims=True))

            total = jax.lax.fori_loop(
                0, 32, tail_body, jnp.zeros((1, BLK), jnp.float32), unroll=2)
            out_ref[0, 0, :] = total[0]
        k *= 2


@jax.jit
def kernel(pred_waveforms, obs_waveforms):
    pred = pred_waveforms.reshape(NT, NCOL)
    obs = obs_waveforms.reshape(NT, NCOL)
    partial = pl.pallas_call(
        _wasserstein_kernel,
        grid=(NBLK,),
        in_specs=[
            pl.BlockSpec((NT, BLK), lambda i: (0, i)),
            pl.BlockSpec((NT, BLK), lambda i: (0, i)),
        ],
        out_specs=pl.BlockSpec((1, 1, BLK), lambda i: (i, 0, 0)),
        out_shape=jax.ShapeDtypeStruct((NBLK, 1, BLK), jnp.float32),
        scratch_shapes=[
            pltpu.VMEM((NT, BLK), jnp.float32),
            pltpu.VMEM((NT, BLK), jnp.float32),
        ],
    )(pred, obs)
    return jnp.sum(partial) / (NT * NCOL)


# speedup vs baseline: 1.0119x; 1.0048x over previous
import jax
import jax.numpy as jnp
from jax.experimental import pallas as pl
from jax.experimental.pallas import tpu as pltpu

NT = 4096
NCOL = 384 * 3
BLK = 128
NBLK = NCOL // BLK


def _shift(x, j):
    m = x.shape[0]
    if j > 0:
        return jnp.concatenate([x[j:], x[:j]], axis=0)
    j = -j
    return jnp.concatenate([x[m - j:], x[:m - j]], axis=0)


def _uniform_large(x, j, min_down):
    m = x.shape[0]
    x4 = x.reshape(m // (2 * j), 2, j, BLK)
    a = x4[:, 0]
    b = x4[:, 1]
    lo = jnp.minimum(a, b)[:, None]
    hi = jnp.maximum(a, b)[:, None]
    pair = [lo, hi] if min_down else [hi, lo]
    return jnp.concatenate(pair, axis=1).reshape(m, BLK)


def _uniform_small(x, j, masks, min_down):
    bitj = masks[(x.shape[0], j)]
    u = _shift(x, j)
    d = _shift(x, -j)
    if min_down:
        return jnp.where(bitj, jnp.maximum(x, d), jnp.minimum(x, u))
    return jnp.where(bitj, jnp.minimum(x, d), jnp.maximum(x, u))


def _masked_substage(x, j, bitj, keep_min):
    p = jnp.where(bitj, _shift(x, -j), _shift(x, j))
    return jnp.where(keep_min, jnp.minimum(x, p), jnp.maximum(x, p))


def _stage_uniform(x, k, masks, min_down):
    j = k // 2
    while j >= 1:
        if j >= 8:
            x = _uniform_large(x, j, min_down)
        else:
            x = _uniform_small(x, j, masks, min_down)
        j //= 2
    return x


def _tile_sort64(x, masks, ascending):
    for k in (2, 4):
        kbit = masks[("k", k)]
        j = k // 2
        while j >= 1:
            bitj = masks[(64, j)]
            keep_min = (bitj == kbit) if ascending else (bitj != kbit)
            x = _masked_substage(x, j, bitj, keep_min)
            j //= 2
    for k in (8, 16, 32):
        g = 64 // (2 * k)
        x6 = x.reshape(g, 2, k, BLK)
        xa = x6[:, 0].reshape(32, BLK)
        xd = x6[:, 1].reshape(32, BLK)
        xa = _stage_uniform(xa, k, masks, ascending)
        xd = _stage_uniform(xd, k, masks, not ascending)
        x = jnp.concatenate(
            [xa.reshape(g, 1, k, BLK), xd.reshape(g, 1, k, BLK)],
            axis=1).reshape(64, BLK)
    return _stage_uniform(x, 64, masks, ascending)


def _wasserstein_kernel(pred_ref, obs_ref, out_ref, a_s, b_s):
    masks = {}
    for rows in (32, 64):
        it = jax.lax.broadcasted_iota(jnp.int32, (rows, 1), 0)
        for j in (1, 2, 4):
            masks[(rows, j)] = (it & j) != 0
        if rows == 64:
            for k in (2, 4, 8):
                masks[("k", k)] = (it & k) != 0

    def body1(m, carry):
        for src, dst in ((pred_ref, a_s), (obs_ref, b_s)):
            for off, asc in ((m * 128, True), (m * 128 + 64, False)):
                xt = src[pl.ds(off, 64), :]
                dst[pl.ds(off, 64), :] = _tile_sort64(xt, masks, asc)
        return carry

    jax.lax.fori_loop(0, 32, body1, 0, unroll=4)

    def _pair_tail(ref, p, asc):
        off = p * 128
        t0 = ref[pl.ds(off, 64), :]
        t1 = ref[pl.ds(off + 64, 64), :]
        lo = jnp.minimum(t0, t1)
        hi = jnp.maximum(t0, t1)
        x0, x1 = (lo, hi) if asc else (hi, lo)
        ref[pl.ds(off, 64), :] = _stage_uniform(x0, 64, masks, asc)
        ref[pl.ds(off + 64, 64), :] = _stage_uniform(x1, 64, masks, asc)

    k = 128
    while k <= NT:
        if k > 128:
            for ref in (a_s, b_s):
                x = ref[...]
                if k < NT:
                    x6 = x.reshape(NT // (2 * k), 2, k, BLK)
                    xa = x6[:, 0].reshape(NT // 2, BLK)
                    xd = x6[:, 1].reshape(NT // 2, BLK)
                    j = k // 2
                    while j >= 128:
                        xa = _uniform_large(xa, j, True)
                        xd = _uniform_large(xd, j, False)
                        j //= 2
                    x = jnp.concatenate(
                        [xa.reshape(NT // (2 * k), 1, k, BLK),
                         xd.reshape(NT // (2 * k), 1, k, BLK)],
                        axis=1).reshape(NT, BLK)
                else:
                    j = k // 2
                    while j >= 128:
                        x = _uniform_large(x, j, True)
                        j //= 2
                ref[...] = x

        q = k // 128

        if k < NT:
            def tail_body(m, carry, q=q):
                p_asc = (m // q) * (2 * q) + (m % q)
                for ref in (a_s, b_s):
                    _pair_tail(ref, p_asc, True)
                    _pair_tail(ref, p_asc + q, False)
                return carry

            jax.lax.fori_loop(0, 16, tail_body, 0, unroll=4)
        else:
            def tail_body(p, carry):
                off = p * 128
                outs = []
                for ref in (a_s, b_s):
                    t0 = ref[pl.ds(off, 64), :]
                    t1 = ref[pl.ds(off + 64, 64), :]
                    lo = jnp.minimum(t0, t1)
                    hi = jnp.maximum(t0, t1)
                    outs.append((_stage_uniform(lo, 64, masks, True),
                                 _stage_uniform(hi, 64, masks, True)))
                (a0, a1), (b0, b1) = outs
                return (carry
                        + jnp.sum(jnp.abs(a0 - b0), axis=0, keepdims=True)
                        + jnp.sum(jnp.abs(a1 - b1), axis=0, keepdims=True))

            total = jax.lax.fori_loop(
                0, 32, tail_body, jnp.zeros((1, BLK), jnp.float32), unroll=4)
            out_ref[0, 0, :] = total[0]
        k *= 2


@jax.jit
def kernel(pred_waveforms, obs_waveforms):
    pred = pred_waveforms.reshape(NT, NCOL)
    obs = obs_waveforms.reshape(NT, NCOL)
    partial = pl.pallas_call(
        _wasserstein_kernel,
        grid=(NBLK,),
        in_specs=[
            pl.BlockSpec((NT, BLK), lambda i: (0, i)),
            pl.BlockSpec((NT, BLK), lambda i: (0, i)),
        ],
        out_specs=pl.BlockSpec((1, 1, BLK), lambda i: (i, 0, 0)),
        out_shape=jax.ShapeDtypeStruct((NBLK, 1, BLK), jnp.float32),
        scratch_shapes=[
            pltpu.VMEM((NT, BLK), jnp.float32),
            pltpu.VMEM((NT, BLK), jnp.float32),
        ],
    )(pred, obs)
    return jnp.sum(partial) / (NT * NCOL)
